# baseline (speedup 1.0000x reference)
import functools

import jax
import jax.numpy as jnp
from jax.experimental import pallas as pl
from jax.experimental.pallas import tpu as pltpu
from jax.experimental.pallas import tpu_sc as plsc

EPS = 1e-5
TB = 512
GW = 128


def _sc_gather(emb_flat, fm_flat, idx2):
    R = idx2.shape[1]
    D = emb_flat.shape[1]
    mesh = plsc.VectorSubcoreMesh(core_axis_name="core", subcore_axis_name="subcore")

    @functools.partial(
        pl.kernel,
        out_type=[
            jax.ShapeDtypeStruct((R, D), jnp.float32),
            jax.ShapeDtypeStruct((R,), jnp.float32),
        ],
        mesh=mesh,
        compiler_params=pltpu.CompilerParams(use_tc_tiling_on_sc=False),
    )
    def k(emb_hbm, fm_hbm, i_hbm, oemb_hbm, ofm_hbm):
        def body(i_vmem, oe_vmem, of_vmem):
            pltpu.sync_copy(emb_hbm.at[i_vmem.at[0]], oe_vmem)
            pltpu.sync_copy(fm_hbm.at[i_vmem.at[0]], of_vmem)

        pltpu.emit_pipeline(
            body,
            grid=(R // GW,),
            in_specs=[pl.BlockSpec((1, GW), index_map=lambda i: (0, i))],
            out_specs=[
                pl.BlockSpec((GW, D), index_map=lambda i: (i, 0)),
                pl.BlockSpec((GW,), index_map=lambda i: (i,)),
            ],
            core_axis_name=("core", "subcore"),
            dimension_semantics=(pltpu.PARALLEL,),
        )(i_hbm, oemb_hbm, ofm_hbm)

    return k(emb_flat, fm_flat, idx2)


def _tc1_body(emb_ref, xnum_ref, fm_ref, m_ref, w1a_ref, w1b_ref, b1_ref, b3_ref,
              h1_ref, fmsum_ref, s_ref, ss_ref):
    emb = emb_ref[...]
    s16 = jax.lax.dot(emb, m_ref[...], precision=jax.lax.Precision.HIGHEST)
    sq = jnp.sum(emb * emb, axis=1)
    fm2 = 0.5 * (jnp.sum(s16 * s16, axis=1) - sq)
    fm1 = jnp.sum(fm_ref[...], axis=1)
    fmsum_ref[...] = (fm1 + fm2 + b3_ref[0, 0])[:, None]

    h = jnp.dot(emb.astype(jnp.bfloat16), w1a_ref[...],
                preferred_element_type=jnp.float32)
    h = h + jax.lax.dot(xnum_ref[...], w1b_ref[...],
                        precision=jax.lax.Precision.HIGHEST)
    h = h + b1_ref[...]
    h1_ref[...] = h

    @pl.when(pl.program_id(0) == 0)
    def _():
        s_ref[...] = jnp.zeros_like(s_ref)
        ss_ref[...] = jnp.zeros_like(ss_ref)

    s_ref[...] += jnp.sum(h, axis=0, keepdims=True)
    ss_ref[...] += jnp.sum(h * h, axis=0, keepdims=True)


def _tc2_body(h1_ref, s_ref, ss_ref, g1_ref, be1_ref, w2_ref, b2_ref,
              h2_ref, s2_ref, ss2_ref, *, batch):
    mean = s_ref[...] * (1.0 / batch)
    var = ss_ref[...] * (1.0 / batch) - mean * mean
    inv = g1_ref[...] / jnp.sqrt(var + EPS)
    a = jnp.maximum(h1_ref[...] * inv + (be1_ref[...] - mean * inv), 0.0)
    h = jnp.dot(a.astype(jnp.bfloat16), w2_ref[...],
                preferred_element_type=jnp.float32)
    h = h + b2_ref[...]
    h2_ref[...] = h

    @pl.when(pl.program_id(0) == 0)
    def _():
        s2_ref[...] = jnp.zeros_like(s2_ref)
        ss2_ref[...] = jnp.zeros_like(ss2_ref)

    s2_ref[...] += jnp.sum(h, axis=0, keepdims=True)
    ss2_ref[...] += jnp.sum(h * h, axis=0, keepdims=True)


def _tc3_body(h2_ref, s2_ref, ss2_ref, g2_ref, be2_ref, w3_ref, fmsum_ref,
              out_ref, *, batch):
    mean = s2_ref[...] * (1.0 / batch)
    var = ss2_ref[...] * (1.0 / batch) - mean * mean
    inv = g2_ref[...] / jnp.sqrt(var + EPS)
    a = jnp.maximum(h2_ref[...] * inv + (be2_ref[...] - mean * inv), 0.0)
    dnn = jnp.sum(a * w3_ref[...], axis=1)
    logit = dnn + fmsum_ref[:, 0]
    out_ref[...] = jax.nn.sigmoid(logit)[:, None]


def kernel(x_cat, x_num, emb_tables, fm_table, offsets,
           W1, b1, g1, be1, W2, b2, g2, be2, W3, b3):
    B, F = x_cat.shape
    _, V, D = emb_tables.shape
    NUM = x_num.shape[1]
    H = W1.shape[1]
    R = B * F
    NB = B // TB

    idx2 = (x_cat.astype(jnp.int32) + offsets[None, :].astype(jnp.int32)
            ).reshape(1, R)
    emb_flat = emb_tables.reshape(F * V, D)

    emb_rows, fm_rows = _sc_gather(emb_flat, fm_table.reshape(F * V), idx2)
    emb2d = emb_rows.reshape(B, F * D)
    fm2d = fm_rows.reshape(B, F)

    m_fold = jnp.tile(jnp.eye(D, dtype=jnp.float32), (F, 1))
    w1a = W1[:F * D].astype(jnp.bfloat16)
    w1b = W1[F * D:]
    w2 = W2.astype(jnp.bfloat16)
    b1r = b1.reshape(1, H)
    b2r = b2.reshape(1, H)
    g1r = g1.reshape(1, H)
    be1r = be1.reshape(1, H)
    g2r = g2.reshape(1, H)
    be2r = be2.reshape(1, H)
    w3r = W3.reshape(1, H)
    b3r = b3.reshape(1, 1)

    const = lambda shape: pl.BlockSpec(shape, lambda i: (0, 0))
    row = lambda shape: pl.BlockSpec(shape, lambda i: (i, 0))

    f32 = jnp.float32
    h1, fmsum, s1, ss1 = pl.pallas_call(
        _tc1_body,
        grid=(NB,),
        in_specs=[
            row((TB, F * D)), row((TB, NUM)), row((TB, F)),
            const((F * D, D)), const((F * D, H)), const((NUM, H)),
            const((1, H)), const((1, 1)),
        ],
        out_specs=[row((TB, H)), row((TB, 1)), const((1, H)), const((1, H))],
        out_shape=[
            jax.ShapeDtypeStruct((B, H), f32),
            jax.ShapeDtypeStruct((B, 1), f32),
            jax.ShapeDtypeStruct((1, H), f32),
            jax.ShapeDtypeStruct((1, H), f32),
        ],
    )(emb2d, x_num, fm2d, m_fold, w1a, w1b, b1r, b3r)

    h2, s2, ss2 = pl.pallas_call(
        functools.partial(_tc2_body, batch=B),
        grid=(NB,),
        in_specs=[
            row((TB, H)), const((1, H)), const((1, H)),
            const((1, H)), const((1, H)), const((H, H)), const((1, H)),
        ],
        out_specs=[row((TB, H)), const((1, H)), const((1, H))],
        out_shape=[
            jax.ShapeDtypeStruct((B, H), f32),
            jax.ShapeDtypeStruct((1, H), f32),
            jax.ShapeDtypeStruct((1, H), f32),
        ],
    )(h1, s1, ss1, g1r, be1r, w2, b2r)

    out2d = pl.pallas_call(
        functools.partial(_tc3_body, batch=B),
        grid=(NB,),
        in_specs=[
            row((TB, H)), const((1, H)), const((1, H)),
            const((1, H)), const((1, H)), const((1, H)), row((TB, 1)),
        ],
        out_specs=row((TB, 1)),
        out_shape=jax.ShapeDtypeStruct((B, 1), f32),
    )(h2, s2, ss2, g2r, be2r, w3r, fmsum)

    return out2d.reshape(B)
